# Initial kernel scaffold; baseline (speedup 1.0000x reference)
import jax
import jax.numpy as jnp
from jax.experimental import pallas as pl


def kernel(x, salt_idx, pepper_idx):
    raise NotImplementedError("write your pallas kernel here")


# speedup vs baseline: 1.6581x; 1.6581x over previous
import functools

import jax
import jax.numpy as jnp
from jax import lax
from jax.experimental import pallas as pl
from jax.experimental.pallas import tpu as pltpu
from jax.experimental.pallas import tpu_sc as plsc

_NC = 2
_NS = 16
_NW = _NC * _NS
_LANES = 16
_CHUNK = 128


def _pad_indices(idx, k):
    total = _NW * k * _CHUNK
    pad = total - idx.shape[0]
    idx_p = jnp.concatenate([idx, jnp.broadcast_to(idx[:1], (pad,))])
    return idx_p.reshape(_NW, k, _CHUNK)


def _copy_body(x_ref, o_ref):
    o_ref[...] = x_ref[...]


def _tc_copy(flat, rows, block_rows):
    x2 = flat.reshape(rows, 1024)
    y2 = pl.pallas_call(
        _copy_body,
        out_shape=jax.ShapeDtypeStruct((rows, 1024), jnp.float32),
        grid=(rows // block_rows,),
        in_specs=[pl.BlockSpec((block_rows, 1024), lambda i: (i, 0))],
        out_specs=pl.BlockSpec((block_rows, 1024), lambda i: (i, 0)),
    )(x2)
    return y2.reshape(flat.shape)


def _fill_const(ref, value):
    vec = jnp.full((_LANES,), value, jnp.float32)
    for i in range(_CHUNK // _LANES):
        ref[pl.ds(i * _LANES, _LANES)] = vec


@functools.lru_cache(maxsize=None)
def _make_scatter(k_salt, k_pepper):
    mesh = plsc.VectorSubcoreMesh(
        core_axis_name="c", subcore_axis_name="s",
        num_cores=_NC, num_subcores=_NS)

    @functools.partial(
        pl.kernel,
        out_type=(),
        mesh=mesh,
        scratch_types=[
            pltpu.VMEM((k_salt, _CHUNK), jnp.int32),
            pltpu.VMEM((k_pepper, _CHUNK), jnp.int32),
            pltpu.VMEM((_CHUNK,), jnp.float32),
            pltpu.VMEM((_CHUNK,), jnp.float32),
            pltpu.SemaphoreType.DMA,
        ],
    )
    def scatter(salt_hbm, pepper_hbm, y_ref, salt_v, pepper_v, ones_v,
                zeros_v, sem):
        wid = lax.axis_index("s") * _NC + lax.axis_index("c")
        pltpu.sync_copy(salt_hbm.at[wid], salt_v)
        pltpu.sync_copy(pepper_hbm.at[wid], pepper_v)
        _fill_const(ones_v, 1.0)
        _fill_const(zeros_v, 0.0)

        def fire_salt(j, carry):
            pltpu.make_async_copy(ones_v, y_ref.at[salt_v.at[j]], sem).start()
            return carry

        def fire_pepper(j, carry):
            pltpu.make_async_copy(zeros_v, y_ref.at[pepper_v.at[j]], sem).start()
            return carry

        def drain_salt(j, carry):
            pltpu.make_async_copy(ones_v, y_ref.at[salt_v.at[j]], sem).wait()
            return carry

        def drain_pepper(j, carry):
            pltpu.make_async_copy(zeros_v, y_ref.at[pepper_v.at[j]], sem).wait()
            return carry

        lax.fori_loop(0, k_salt, fire_salt, 0)
        lax.fori_loop(0, k_pepper, fire_pepper, 0)
        lax.fori_loop(0, k_salt, drain_salt, 0)
        lax.fori_loop(0, k_pepper, drain_pepper, 0)

    return scatter


def kernel(x, salt_idx, pepper_idx):
    n = x.size
    flat = x.reshape(n)
    rows = n // 1024
    block_rows = 1024 if rows % 1024 == 0 else rows
    y = _tc_copy(flat, rows, block_rows)

    per_pass = _NW * _CHUNK
    k_salt = -(-salt_idx.shape[0] // per_pass)
    k_pepper = -(-pepper_idx.shape[0] // per_pass)
    salt_p = _pad_indices(salt_idx, k_salt)
    pepper_p = _pad_indices(pepper_idx, k_pepper)

    y_ref = jax.new_ref(y)
    _make_scatter(k_salt, k_pepper)(salt_p, pepper_p, y_ref)
    return y_ref[...].reshape(x.shape)


# speedup vs baseline: 1.6667x; 1.0052x over previous
import functools

import jax
import jax.numpy as jnp
from jax import lax
from jax.experimental import pallas as pl
from jax.experimental.pallas import tpu as pltpu
from jax.experimental.pallas import tpu_sc as plsc

_NC = 2
_NS = 16
_NW = _NC * _NS
_LANES = 16
_CHUNK = 128


def _pad_indices(idx, k):
    total = _NW * k * _CHUNK
    pad = total - idx.shape[0]
    idx_p = jnp.concatenate([idx, jnp.broadcast_to(idx[:1], (pad,))])
    return idx_p.reshape(_NW, k * _CHUNK)


def _copy_body(x_ref, o_ref):
    o_ref[...] = x_ref[...]


def _tc_copy(flat, rows, block_rows):
    x2 = flat.reshape(rows, 1024)
    y2 = pl.pallas_call(
        _copy_body,
        out_shape=jax.ShapeDtypeStruct((rows, 1024), jnp.float32),
        grid=(rows // block_rows,),
        in_specs=[pl.BlockSpec((block_rows, 1024), lambda i: (i, 0))],
        out_specs=pl.BlockSpec((block_rows, 1024), lambda i: (i, 0)),
    )(x2)
    return y2.reshape(flat.shape)


def _fill_const(ref, n, value):
    vec = jnp.full((_LANES,), value, jnp.float32)

    def body(i, carry):
        ref[pl.ds(i * _LANES, _LANES)] = vec
        return carry

    lax.fori_loop(0, n // _LANES, body, 0)


@functools.lru_cache(maxsize=None)
def _make_scatter(k_salt, k_pepper):
    mesh = plsc.VectorSubcoreMesh(
        core_axis_name="c", subcore_axis_name="s",
        num_cores=_NC, num_subcores=_NS)

    @functools.partial(
        pl.kernel,
        out_type=(),
        mesh=mesh,
        scratch_types=[
            pltpu.VMEM((k_salt * _CHUNK,), jnp.int32),
            pltpu.VMEM((k_pepper * _CHUNK,), jnp.int32),
            pltpu.VMEM((k_salt * _CHUNK,), jnp.float32),
            pltpu.VMEM((k_pepper * _CHUNK,), jnp.float32),
            pltpu.SemaphoreType.DMA,
        ],
    )
    def scatter(salt_hbm, pepper_hbm, y_ref, salt_v, pepper_v, ones_v,
                zeros_v, sem):
        wid = lax.axis_index("s") * _NC + lax.axis_index("c")
        pltpu.sync_copy(salt_hbm.at[wid], salt_v)
        pltpu.sync_copy(pepper_hbm.at[wid], pepper_v)
        _fill_const(ones_v, k_salt * _CHUNK, 1.0)
        _fill_const(zeros_v, k_pepper * _CHUNK, 0.0)

        cp_s = pltpu.make_async_copy(ones_v, y_ref.at[salt_v], sem)
        cp_p = pltpu.make_async_copy(zeros_v, y_ref.at[pepper_v], sem)
        cp_s.start()
        cp_p.start()
        cp_s.wait()
        cp_p.wait()

    return scatter


def kernel(x, salt_idx, pepper_idx):
    n = x.size
    flat = x.reshape(n)
    rows = n // 1024
    block_rows = 1024 if rows % 1024 == 0 else rows
    y = _tc_copy(flat, rows, block_rows)

    per_pass = _NW * _CHUNK
    k_salt = -(-salt_idx.shape[0] // per_pass)
    k_pepper = -(-pepper_idx.shape[0] // per_pass)
    salt_p = _pad_indices(salt_idx, k_salt)
    pepper_p = _pad_indices(pepper_idx, k_pepper)

    y_ref = jax.new_ref(y)
    _make_scatter(k_salt, k_pepper)(salt_p, pepper_p, y_ref)
    return y_ref[...].reshape(x.shape)


# speedup vs baseline: 1.9943x; 1.1965x over previous
import functools

import jax
import jax.numpy as jnp
from jax import lax
from jax.experimental import pallas as pl
from jax.experimental.pallas import tpu as pltpu
from jax.experimental.pallas import tpu_sc as plsc

_NC = 1
_NS = 16
_NW = _NC * _NS
_L = 16

_WPC = 24
_NWIN = _NC * _WPC
_MYSC_CAP = 9472
_WIN_CAP = 768
_CHUNK = 128
_WROWS = _WIN_CAP // _CHUNK


def _pad_to(idx, m, n):
    pad = _NW * m - idx.shape[0]
    return jnp.concatenate(
        [idx, jnp.full((pad,), n, jnp.int32)]).reshape(_NW, m)


def _prefill(ref, cap, vec):
    def body(i, carry):
        ref[pl.ds(i * _L, _L)] = vec
        return carry

    lax.fori_loop(0, cap // _L, body, 0)


@functools.lru_cache(maxsize=None)
def _make_kernel(n, m):
    ws = n // _NWIN
    wslice = ws // _NS
    half = n // _NC
    nv = m // _L
    mesh = plsc.VectorSubcoreMesh(
        core_axis_name="c", subcore_axis_name="s",
        num_cores=_NC, num_subcores=_NS)

    @functools.partial(
        pl.kernel,
        out_type=jax.ShapeDtypeStruct((n,), jnp.float32),
        mesh=mesh,
        compiler_params=pltpu.CompilerParams(needs_layout_passes=False),
        scratch_types=[
            pltpu.VMEM_SHARED((ws + _L,), jnp.float32),
            pltpu.VMEM((m,), jnp.int32),
            pltpu.VMEM((m,), jnp.int32),
            pltpu.VMEM((_MYSC_CAP,), jnp.int32),
            pltpu.VMEM((_MYSC_CAP,), jnp.int32),
            pltpu.VMEM((_WROWS, _CHUNK), jnp.int32),
            pltpu.VMEM((_WROWS, _CHUNK), jnp.int32),
            pltpu.VMEM((_CHUNK,), jnp.float32),
            pltpu.VMEM((_CHUNK,), jnp.float32),
            pltpu.SemaphoreType.DMA,
        ],
    )
    def run(x_hbm, salt_hbm, pepper_hbm, out_hbm, win_sp, salt_v, pepper_v,
            mys_v, myp_v, wsalt_v, wpep_v, ones_v, zeros_v, sem):
        cid = lax.axis_index("c")
        sid = lax.axis_index("s")
        wid = sid * _NC + cid
        lo = cid * half

        pltpu.sync_copy(salt_hbm.at[wid], salt_v)
        pltpu.sync_copy(pepper_hbm.at[wid], pepper_v)

        sentinel = jnp.full((_L,), n, jnp.int32)
        garbage = jnp.full((_L,), ws, jnp.int32) + sid
        _prefill(mys_v, _MYSC_CAP, sentinel)
        _prefill(myp_v, _MYSC_CAP, sentinel)
        _prefill(ones_v, _CHUNK, jnp.full((_L,), 1.0, jnp.float32))
        _prefill(zeros_v, _CHUNK, jnp.full((_L,), 0.0, jnp.float32))

        def _prefill2d(ref, vec):
            def body(t, carry):
                ref[t >> 3, pl.ds((t & 7) * _L, _L)] = vec
                return carry

            lax.fori_loop(0, _WROWS * (_CHUNK // _L), body, 0)

        lane = lax.iota(jnp.int32, _L)

        def compact(src_v, n_vregs, dst_store, base, span, keep, fill, cap):
            def body(i, off):
                v = src_v[pl.ds(i * _L, _L)]
                rel = v - base
                msk = (rel >= 0) & (rel < span)
                mi = jnp.where(msk, 1, 0).astype(jnp.int32)
                ranks = plsc.cumsum(mi) - 1
                dest = jnp.where(msk, off + ranks, cap - _L + lane)
                val = jnp.where(msk, v if keep else rel, fill)
                dst_store(dest, val)
                return off + jnp.sum(mi).astype(jnp.int32)

            return lax.fori_loop(0, n_vregs, body, jnp.int32(0))

        def store1d(dst_v):
            return lambda dest, val: plsc.store_scatter(dst_v, [dest], val)

        def store2d(dst_v):
            return lambda dest, val: plsc.store_scatter(
                dst_v, [dest >> 7, dest & (_CHUNK - 1)], val)

        n_mys = compact(salt_v, nv, store1d(mys_v), lo, half, True,
                        sentinel, _MYSC_CAP)
        n_myp = compact(pepper_v, nv, store1d(myp_v), lo, half, True,
                        sentinel, _MYSC_CAP)

        def compact_window(src_v, cnt, dst_v, wlo):
            return compact(src_v, (cnt + _L - 1) // _L, store2d(dst_v),
                           wlo, ws, False, garbage, _WIN_CAP)

        def window_pass(j, carry):
            wlo = lo + j * ws
            pltpu.sync_copy(
                x_hbm.at[pl.ds(wlo + sid * wslice, wslice)],
                win_sp.at[pl.ds(sid * wslice, wslice)])

            _prefill2d(wsalt_v, garbage)
            _prefill2d(wpep_v, garbage)
            compact_window(mys_v, n_mys, wsalt_v, wlo)
            compact_window(myp_v, n_myp, wpep_v, wlo)
            plsc.subcore_barrier()

            for r in range(_WROWS):
                pltpu.sync_copy(ones_v, win_sp.at[wsalt_v.at[r]])
                pltpu.sync_copy(zeros_v, win_sp.at[wpep_v.at[r]])
            plsc.subcore_barrier()

            pltpu.sync_copy(
                win_sp.at[pl.ds(sid * wslice, wslice)],
                out_hbm.at[pl.ds(wlo + sid * wslice, wslice)])
            plsc.subcore_barrier()
            return carry

        lax.fori_loop(0, _WPC, window_pass, 0)

    return run


def kernel(x, salt_idx, pepper_idx):
    n = x.size
    flat = x.reshape(n)
    per = -(-salt_idx.shape[0] // (_NW * _L)) * _L
    salt_p = _pad_to(salt_idx, per, n)
    pepper_p = _pad_to(pepper_idx, per, n)
    out = _make_kernel(n, per)(flat, salt_p, pepper_p)
    return out.reshape(x.shape)


# speedup vs baseline: 3.6689x; 1.8397x over previous
import functools

import jax
import jax.numpy as jnp
from jax import lax
from jax.experimental import pallas as pl
from jax.experimental.pallas import tpu as pltpu
from jax.experimental.pallas import tpu_sc as plsc

_NC = 2
_NS = 16
_L = 16

_WPC = 12
_NWIN = _NC * _WPC
_MYSC_CAP = 4864
_WIN_CAP = 640
_CHUNK = 128
_WROWS = _WIN_CAP // _CHUNK


def _pad_to(idx, m, n):
    pad = _NS * m - idx.shape[0]
    return jnp.concatenate(
        [idx, jnp.full((pad,), n, jnp.int32)]).reshape(_NS, m)


def _prefill(ref, cap, vec):
    def body(i, carry):
        ref[pl.ds(i * _L, _L)] = vec
        return carry

    lax.fori_loop(0, cap // _L, body, 0)


@functools.lru_cache(maxsize=None)
def _make_kernel(n, m):
    ws = n // _NWIN
    wslice = ws // _NS
    half = n // _NC
    nv = m // _L
    mesh = plsc.VectorSubcoreMesh(
        core_axis_name="c", subcore_axis_name="s",
        num_cores=_NC, num_subcores=_NS)

    @functools.partial(
        pl.kernel,
        out_type=jax.ShapeDtypeStruct((n,), jnp.float32),
        mesh=mesh,
        compiler_params=pltpu.CompilerParams(needs_layout_passes=False),
        scratch_types=[
            pltpu.VMEM_SHARED((ws + _L,), jnp.float32),
            pltpu.VMEM((m,), jnp.int32),
            pltpu.VMEM((m,), jnp.int32),
            pltpu.VMEM((_MYSC_CAP,), jnp.int32),
            pltpu.VMEM((_MYSC_CAP,), jnp.int32),
            pltpu.VMEM((_WROWS, _CHUNK), jnp.int32),
            pltpu.VMEM((_WROWS, _CHUNK), jnp.int32),
            pltpu.VMEM((_CHUNK,), jnp.float32),
            pltpu.VMEM((_CHUNK,), jnp.float32),
            pltpu.SemaphoreType.DMA,
        ],
    )
    def run(x_hbm, salt_hbm, pepper_hbm, out_hbm, win_sp, salt_v, pepper_v,
            mys_v, myp_v, wsalt_v, wpep_v, ones_v, zeros_v, sem):
        cid = lax.axis_index("c")
        sid = lax.axis_index("s")
        lo = cid * half

        pltpu.sync_copy(salt_hbm.at[sid], salt_v)
        pltpu.sync_copy(pepper_hbm.at[sid], pepper_v)

        sentinel = jnp.full((_L,), n, jnp.int32)
        garbage = jnp.full((_L,), ws, jnp.int32) + sid
        _prefill(mys_v, _MYSC_CAP, sentinel)
        _prefill(myp_v, _MYSC_CAP, sentinel)
        _prefill(ones_v, _CHUNK, jnp.full((_L,), 1.0, jnp.float32))
        _prefill(zeros_v, _CHUNK, jnp.full((_L,), 0.0, jnp.float32))

        def _prefill2d(ref, vec):
            def body(t, carry):
                ref[t >> 3, pl.ds((t & 7) * _L, _L)] = vec
                return carry

            lax.fori_loop(0, _WROWS * (_CHUNK // _L), body, 0)

        lane = lax.iota(jnp.int32, _L)

        def compact(src_v, n_vregs, dst_store, base, span, keep, fill, cap):
            def body(i, off):
                v = src_v[pl.ds(i * _L, _L)]
                rel = v - base
                msk = (rel >= 0) & (rel < span)
                mi = jnp.where(msk, 1, 0).astype(jnp.int32)
                ranks = plsc.cumsum(mi) - 1
                dest = jnp.where(msk, off + ranks, cap - _L + lane)
                val = jnp.where(msk, v if keep else rel, fill)
                dst_store(dest, val)
                return off + jnp.sum(mi).astype(jnp.int32)

            return lax.fori_loop(0, n_vregs, body, jnp.int32(0))

        def store1d(dst_v):
            return lambda dest, val: plsc.store_scatter(dst_v, [dest], val)

        def store2d(dst_v):
            return lambda dest, val: plsc.store_scatter(
                dst_v, [dest >> 7, dest & (_CHUNK - 1)], val)

        n_mys = compact(salt_v, nv, store1d(mys_v), lo, half, True,
                        sentinel, _MYSC_CAP)
        n_myp = compact(pepper_v, nv, store1d(myp_v), lo, half, True,
                        sentinel, _MYSC_CAP)

        def compact_window(src_v, cnt, dst_v, wlo):
            return compact(src_v, (cnt + _L - 1) // _L, store2d(dst_v),
                           wlo, ws, False, garbage, _WIN_CAP)

        def window_pass(j, carry):
            wlo = lo + j * ws
            pltpu.sync_copy(
                x_hbm.at[pl.ds(wlo + sid * wslice, wslice)],
                win_sp.at[pl.ds(sid * wslice, wslice)])

            _prefill2d(wsalt_v, garbage)
            _prefill2d(wpep_v, garbage)
            compact_window(mys_v, n_mys, wsalt_v, wlo)
            compact_window(myp_v, n_myp, wpep_v, wlo)
            plsc.subcore_barrier()

            for r in range(_WROWS):
                pltpu.sync_copy(ones_v, win_sp.at[wsalt_v.at[r]])
                pltpu.sync_copy(zeros_v, win_sp.at[wpep_v.at[r]])
            plsc.subcore_barrier()

            pltpu.sync_copy(
                win_sp.at[pl.ds(sid * wslice, wslice)],
                out_hbm.at[pl.ds(wlo + sid * wslice, wslice)])
            plsc.subcore_barrier()
            return carry

        lax.fori_loop(0, _WPC, window_pass, 0)

    return run


def kernel(x, salt_idx, pepper_idx):
    n = x.size
    flat = x.reshape(n)
    per = -(-salt_idx.shape[0] // (_NS * _L)) * _L
    salt_p = _pad_to(salt_idx, per, n)
    pepper_p = _pad_to(pepper_idx, per, n)
    out = _make_kernel(n, per)(flat, salt_p, pepper_p)
    return out.reshape(x.shape)


# speedup vs baseline: 4.3139x; 1.1758x over previous
import functools

import jax
import jax.numpy as jnp
from jax import lax
from jax.experimental import pallas as pl
from jax.experimental.pallas import tpu as pltpu
from jax.experimental.pallas import tpu_sc as plsc

_NC = 2
_NS = 16
_L = 16

_WPC = 12
_NWIN = _NC * _WPC
_MYSC_CAP = 4864
_WIN_CAP = 640
_CHUNK = 128
_WROWS = _WIN_CAP // _CHUNK


def _pad_to(idx, m, n):
    pad = _NS * m - idx.shape[0]
    return jnp.concatenate(
        [idx, jnp.full((pad,), n, jnp.int32)]).reshape(_NS, m)


def _prefill(ref, cap, vec):
    def body(i, carry):
        ref[pl.ds(i * _L, _L)] = vec
        return carry

    lax.fori_loop(0, cap // _L, body, 0)


@functools.lru_cache(maxsize=None)
def _make_kernel(n, m):
    ws = n // _NWIN
    wslice = ws // _NS
    half = n // _NC
    nv = m // _L
    mesh = plsc.VectorSubcoreMesh(
        core_axis_name="c", subcore_axis_name="s",
        num_cores=_NC, num_subcores=_NS)

    @functools.partial(
        pl.kernel,
        out_type=jax.ShapeDtypeStruct((n,), jnp.float32),
        mesh=mesh,
        compiler_params=pltpu.CompilerParams(needs_layout_passes=False),
        scratch_types=[
            pltpu.VMEM_SHARED((ws + _L,), jnp.float32),
            pltpu.VMEM((m,), jnp.int32),
            pltpu.VMEM((m,), jnp.int32),
            pltpu.VMEM((_MYSC_CAP,), jnp.int32),
            pltpu.VMEM((_MYSC_CAP,), jnp.int32),
            pltpu.VMEM((_WROWS, _CHUNK), jnp.int32),
            pltpu.VMEM((_WROWS, _CHUNK), jnp.int32),
            pltpu.VMEM((_CHUNK,), jnp.float32),
            pltpu.VMEM((_CHUNK,), jnp.float32),
            pltpu.SemaphoreType.DMA,
            pltpu.SemaphoreType.DMA,
        ],
    )
    def run(x_hbm, salt_hbm, pepper_hbm, out_hbm, win_sp, salt_v, pepper_v,
            mys_v, myp_v, wsalt_v, wpep_v, ones_v, zeros_v, sem, sem2):
        cid = lax.axis_index("c")
        sid = lax.axis_index("s")
        lo = cid * half

        pltpu.sync_copy(salt_hbm.at[sid], salt_v)
        pltpu.sync_copy(pepper_hbm.at[sid], pepper_v)

        sentinel = jnp.full((_L,), n, jnp.int32)
        garbage = jnp.full((_L,), ws, jnp.int32) + sid
        _prefill(mys_v, _MYSC_CAP, sentinel)
        _prefill(myp_v, _MYSC_CAP, sentinel)
        _prefill(ones_v, _CHUNK, jnp.full((_L,), 1.0, jnp.float32))
        _prefill(zeros_v, _CHUNK, jnp.full((_L,), 0.0, jnp.float32))

        def _prefill2d(ref, vec):
            def body(t, carry):
                ref[t >> 3, pl.ds((t & 7) * _L, _L)] = vec
                return carry

            lax.fori_loop(0, _WROWS * (_CHUNK // _L), body, 0)

        lane = lax.iota(jnp.int32, _L)

        def compact(src_v, n_vregs, dst_store, base, span, keep, fill, cap):
            def body(i, off):
                v = src_v[pl.ds(i * _L, _L)]
                rel = v - base
                msk = (rel >= 0) & (rel < span)
                mi = jnp.where(msk, 1, 0).astype(jnp.int32)
                cs = plsc.cumsum(mi)
                ranks = cs - 1
                dest = jnp.where(msk, off + ranks, cap - _L + lane)
                val = jnp.where(msk, v if keep else rel, fill)
                dst_store(dest, val)
                return off + cs[_L - 1]

            return lax.fori_loop(0, n_vregs, body, jnp.int32(0))

        def store1d(dst_v):
            return lambda dest, val: plsc.store_scatter(dst_v, [dest], val)

        def store2d(dst_v):
            return lambda dest, val: plsc.store_scatter(
                dst_v, [dest >> 7, dest & (_CHUNK - 1)], val)

        n_mys = compact(salt_v, nv, store1d(mys_v), lo, half, True,
                        sentinel, _MYSC_CAP)
        n_myp = compact(pepper_v, nv, store1d(myp_v), lo, half, True,
                        sentinel, _MYSC_CAP)

        def compact_window(src_v, cnt, dst_v, wlo):
            return compact(src_v, (cnt + _L - 1) // _L, store2d(dst_v),
                           wlo, ws, False, garbage, _WIN_CAP)

        def window_pass(j, carry):
            wlo = lo + j * ws
            load_cp = pltpu.make_async_copy(
                x_hbm.at[pl.ds(wlo + sid * wslice, wslice)],
                win_sp.at[pl.ds(sid * wslice, wslice)], sem)
            load_cp.start()

            _prefill2d(wsalt_v, garbage)
            _prefill2d(wpep_v, garbage)
            compact_window(mys_v, n_mys, wsalt_v, wlo)
            compact_window(myp_v, n_myp, wpep_v, wlo)
            load_cp.wait()
            plsc.subcore_barrier()

            cps = []
            for r in range(_WROWS):
                cps.append(pltpu.make_async_copy(
                    ones_v, win_sp.at[wsalt_v.at[r]], sem2))
                cps.append(pltpu.make_async_copy(
                    zeros_v, win_sp.at[wpep_v.at[r]], sem2))
            for cp in cps:
                cp.start()
            for cp in cps:
                cp.wait()
            plsc.subcore_barrier()

            pltpu.sync_copy(
                win_sp.at[pl.ds(sid * wslice, wslice)],
                out_hbm.at[pl.ds(wlo + sid * wslice, wslice)])
            return carry

        lax.fori_loop(0, _WPC, window_pass, 0)

    return run


def kernel(x, salt_idx, pepper_idx):
    n = x.size
    flat = x.reshape(n)
    per = -(-salt_idx.shape[0] // (_NS * _L)) * _L
    salt_p = _pad_to(salt_idx, per, n)
    pepper_p = _pad_to(pepper_idx, per, n)
    out = _make_kernel(n, per)(flat, salt_p, pepper_p)
    return out.reshape(x.shape)
